# baseline (speedup 1.0000x reference)
import jax
import jax.numpy as jnp
from jax import lax
from jax.experimental import pallas as pl

N = 128
E = 16
D = 256
H = 8
DH = D // H


def _b(a):
    return a.astype(jnp.bfloat16)


def _dotT(a, b):
    return lax.dot_general(_b(a), _b(b), (((1,), (1,)), ((), ())),
                           preferred_element_type=jnp.float32)


def _dot0(a, b):
    return lax.dot_general(_b(a), _b(b), (((0,), (0,)), ((), ())),
                           preferred_element_type=jnp.float32)


def _dot(a, b):
    return lax.dot_general(_b(a), _b(b), (((1,), (0,)), ((), ())),
                           preferred_element_type=jnp.float32)


def _fused_kernel(x_ref, inc_ref, ea_ref, wlin_ref, wedge_ref, wproj_ref,
                  bproj_ref, wout_ref, bout_ref, out_ref):
    G = 4
    qmask = ((lax.broadcasted_iota(jnp.int32, (G * DH, G * N), 0) // DH) ==
             (lax.broadcasted_iota(jnp.int32, (G * DH, G * N), 1) // N)
             ).astype(jnp.bfloat16)

    xv = x_ref[...]
    Bf = (inc_ref[...] != 0).astype(jnp.float32)
    We = _dotT(ea_ref[...], wedge_ref[...])

    scale = 1.0 / (DH ** 0.5)
    bT = bproj_ref[...].reshape(3 * D, 1)
    keT = _dotT(wproj_ref[D:2 * D, :], We) + bT[D:2 * D]

    WhT = _dotT(wlin_ref[...], xv)
    Wh = WhT.T
    PT = _dot(wproj_ref[...], WhT)
    qT = (PT[0:D, :] + bT[0:D]) * scale
    kxT = PT[D:2 * D, :]
    vvT = PT[2 * D:3 * D, :] + bT[2 * D:3 * D]

    ax_groups = []
    qke_groups = []
    for g in range(2):
        gs = slice(g * G * DH, (g + 1) * G * DH)
        qblk = _b(jnp.concatenate([qT[gs, :]] * G, axis=1)) * qmask
        ax_groups.append(_dot0(qblk, kxT[gs, :]))
        qke_groups.append(_dot0(keT[gs, :], qblk))
    ax_all = jnp.exp(jnp.concatenate(ax_groups, axis=0))

    qke_all = jnp.concatenate(qke_groups, axis=1)
    Bf_lane = jnp.concatenate([Bf] * H, axis=1)
    ae_all = jnp.where(Bf_lane > 0.0, jnp.exp(qke_all), 0.0)
    S_all = _dot0(ae_all, Bf)

    w_all = ax_all * S_all

    ones_row = jnp.ones((1, N), dtype=jnp.float32)
    sumvT = jnp.sum(vvT, axis=1, keepdims=True)
    head_outs = []
    for h in range(H):
        sl = slice(h * DH, (h + 1) * DH)
        w = w_all[h * N:(h + 1) * N, :]
        vext = jnp.concatenate([vvT[sl, :], ones_row], axis=0)
        neT = _dotT(vext, w)
        den = neT[DH:DH + 1, :]
        fb = (den <= 0.0).astype(jnp.float32)
        rden = 1.0 / (den + float(N) * fb)
        head_outs.append((neT[0:DH, :] + fb * sumvT[sl]) * rden)
    outhT = jnp.concatenate(head_outs, axis=0)

    out = lax.dot_general(_b(outhT), _b(wout_ref[...]),
                          (((0,), (1,)), ((), ())),
                          preferred_element_type=jnp.float32)
    out = out + bout_ref[...].reshape(1, D)

    any_edge = jnp.max(Bf) > 0.0
    out_ref[...] = jnp.where(any_edge, jnp.maximum(out, 0.0),
                             jnp.maximum(Wh, 0.0))


def kernel(x, incidence, edge_attr, W_lin, W_edge, in_proj_w, in_proj_b,
           out_proj_w, out_proj_b):
    return pl.pallas_call(
        _fused_kernel,
        out_shape=jax.ShapeDtypeStruct((N, D), jnp.float32),
    )(x, incidence, edge_attr, W_lin, W_edge, in_proj_w, in_proj_b,
      out_proj_w, out_proj_b)


# speedup vs baseline: 1.0400x; 1.0400x over previous
import jax
import jax.numpy as jnp
from jax import lax
from jax.experimental import pallas as pl

N = 128
E = 16
D = 256
H = 8
DH = D // H


def _b(a):
    return a.astype(jnp.bfloat16)


def _dotT(a, b):
    return lax.dot_general(_b(a), _b(b), (((1,), (1,)), ((), ())),
                           preferred_element_type=jnp.float32)


def _dot0(a, b):
    return lax.dot_general(_b(a), _b(b), (((0,), (0,)), ((), ())),
                           preferred_element_type=jnp.float32)


def _dot(a, b):
    return lax.dot_general(_b(a), _b(b), (((1,), (0,)), ((), ())),
                           preferred_element_type=jnp.float32)


def _fused_kernel(x_ref, inc_ref, ea_ref, wlin_ref, wedge_ref, wproj_ref,
                  bproj_ref, wout_ref, bout_ref, out_ref):
    G = 4
    qmask = ((lax.broadcasted_iota(jnp.int32, (G * DH, G * N), 0) // DH) ==
             (lax.broadcasted_iota(jnp.int32, (G * DH, G * N), 1) // N)
             ).astype(jnp.bfloat16)

    xv = x_ref[...]
    Bf = (inc_ref[...] != 0).astype(jnp.float32)

    WhT = _dotT(wlin_ref[...], xv)
    Wh = WhT.T
    We = _dotT(ea_ref[...], wedge_ref[...])

    scale = 1.0 / (DH ** 0.5)

    bT = bproj_ref[...].reshape(3 * D, 1)
    PT = _dot(wproj_ref[...], WhT)
    qT = (PT[0:D, :] + bT[0:D]) * scale
    kxT = PT[D:2 * D, :]
    vvT = PT[2 * D:3 * D, :] + bT[2 * D:3 * D]

    keT = _dotT(wproj_ref[D:2 * D, :], We) + bT[D:2 * D]

    ax_groups = []
    qke_groups = []
    for g in range(2):
        gs = slice(g * G * DH, (g + 1) * G * DH)
        qblk = _b(jnp.concatenate([qT[gs, :]] * G, axis=1)) * qmask
        ax_groups.append(_dot0(qblk, kxT[gs, :]))
        qke_groups.append(_dot0(keT[gs, :], qblk))
    ax_all = jnp.exp(jnp.concatenate(ax_groups, axis=0))

    qke_all = jnp.concatenate(qke_groups, axis=1)
    Bf_lane = jnp.concatenate([Bf] * H, axis=1)
    ae_all = jnp.where(Bf_lane > 0.0, jnp.exp(qke_all), 0.0)
    S_all = _dot0(ae_all, Bf)

    w_all = ax_all * S_all

    ones_row = jnp.ones((1, N), dtype=jnp.float32)
    sumvT = jnp.sum(vvT, axis=1, keepdims=True)
    head_outs = []
    for h in range(H):
        sl = slice(h * DH, (h + 1) * DH)
        w = w_all[h * N:(h + 1) * N, :]
        vext = jnp.concatenate([vvT[sl, :], ones_row], axis=0)
        neT = _dotT(vext, w)
        den = neT[DH:DH + 1, :]
        fb = (den <= 0.0).astype(jnp.float32)
        rden = 1.0 / (den + float(N) * fb)
        head_outs.append((neT[0:DH, :] + fb * sumvT[sl]) * rden)
    outhT = jnp.concatenate(head_outs, axis=0)

    out = lax.dot_general(_b(outhT), _b(wout_ref[...]),
                          (((0,), (1,)), ((), ())),
                          preferred_element_type=jnp.float32)
    out = out + bout_ref[...].reshape(1, D)

    any_edge = jnp.max(Bf) > 0.0
    out_ref[...] = jnp.where(any_edge, jnp.maximum(out, 0.0),
                             jnp.maximum(Wh, 0.0))


def kernel(x, incidence, edge_attr, W_lin, W_edge, in_proj_w, in_proj_b,
           out_proj_w, out_proj_b):
    return pl.pallas_call(
        _fused_kernel,
        out_shape=jax.ShapeDtypeStruct((N, D), jnp.float32),
    )(x, incidence, edge_attr, W_lin, W_edge, in_proj_w, in_proj_b,
      out_proj_w, out_proj_b)


# speedup vs baseline: 1.0581x; 1.0174x over previous
import jax
import jax.numpy as jnp
from jax import lax
from jax.experimental import pallas as pl

N = 128
E = 16
D = 256
H = 8
DH = D // H


def _b(a):
    return a.astype(jnp.bfloat16)


def _dotT(a, b):
    return lax.dot_general(_b(a), _b(b), (((1,), (1,)), ((), ())),
                           preferred_element_type=jnp.float32)


def _dot0(a, b):
    return lax.dot_general(_b(a), _b(b), (((0,), (0,)), ((), ())),
                           preferred_element_type=jnp.float32)


def _dot(a, b):
    return lax.dot_general(_b(a), _b(b), (((1,), (0,)), ((), ())),
                           preferred_element_type=jnp.float32)


def _fused_kernel(x_ref, inc_ref, ea_ref, wlin_ref, wedge_ref, wproj_ref,
                  bproj_ref, wout_ref, bout_ref, out_ref):
    G = 8
    qmask = ((lax.broadcasted_iota(jnp.int32, (G * DH, G * N), 0) // DH) ==
             (lax.broadcasted_iota(jnp.int32, (G * DH, G * N), 1) // N)
             ).astype(jnp.bfloat16)

    xv = x_ref[...]
    Bf = (inc_ref[...] != 0).astype(jnp.float32)

    WhT = _dotT(wlin_ref[...], xv)
    Wh = WhT.T
    We = _dotT(ea_ref[...], wedge_ref[...])

    scale = 1.0 / (DH ** 0.5)

    bT = bproj_ref[...].reshape(3 * D, 1)
    PT = _dot(wproj_ref[...], WhT)
    qT = (PT[0:D, :] + bT[0:D]) * scale
    kxT = PT[D:2 * D, :]
    vvT = PT[2 * D:3 * D, :] + bT[2 * D:3 * D]

    keT = _dotT(wproj_ref[D:2 * D, :], We) + bT[D:2 * D]

    qblk = _b(jnp.concatenate([qT] * G, axis=1)) * qmask
    ax_all = jnp.exp(_dot0(qblk, kxT))
    qke_all = _dot0(keT, qblk)
    Bf_lane = jnp.concatenate([Bf] * H, axis=1)
    ae_all = jnp.where(Bf_lane > 0.0, jnp.exp(qke_all), 0.0)
    S_all = _dot0(ae_all, Bf)

    w_all = ax_all * S_all

    ones_row = jnp.ones((1, N), dtype=jnp.float32)
    sumvT = jnp.sum(vvT, axis=1, keepdims=True)
    head_outs = []
    for h in range(H):
        sl = slice(h * DH, (h + 1) * DH)
        w = w_all[h * N:(h + 1) * N, :]
        vext = jnp.concatenate([vvT[sl, :], ones_row], axis=0)
        neT = _dotT(vext, w)
        den = neT[DH:DH + 1, :]
        fb = (den <= 0.0).astype(jnp.float32)
        rden = 1.0 / (den + float(N) * fb)
        head_outs.append((neT[0:DH, :] + fb * sumvT[sl]) * rden)
    outhT = jnp.concatenate(head_outs, axis=0)

    out = lax.dot_general(_b(outhT), _b(wout_ref[...]),
                          (((0,), (1,)), ((), ())),
                          preferred_element_type=jnp.float32)
    out = out + bout_ref[...].reshape(1, D)

    any_edge = jnp.max(Bf) > 0.0
    out_ref[...] = jnp.where(any_edge, jnp.maximum(out, 0.0),
                             jnp.maximum(Wh, 0.0))


def kernel(x, incidence, edge_attr, W_lin, W_edge, in_proj_w, in_proj_b,
           out_proj_w, out_proj_b):
    return pl.pallas_call(
        _fused_kernel,
        out_shape=jax.ShapeDtypeStruct((N, D), jnp.float32),
    )(x, incidence, edge_attr, W_lin, W_edge, in_proj_w, in_proj_b,
      out_proj_w, out_proj_b)
